# baseline (speedup 1.0000x reference)
import functools

import jax
import jax.numpy as jnp
from jax import lax
from jax.experimental import pallas as pl
from jax.experimental.pallas import tpu as pltpu

_NUM_CODES = 8192
_DIM = 256
_TOKENS = 8192
_BLK = 256


def _vq_body(z_ref, w_ref, qz_ref, loss_ref):
    i = pl.program_id(0)
    z = z_ref[...]
    w = w_ref[...]
    zz = jnp.sum(z * z, axis=1, keepdims=True)
    ww = jnp.sum(w * w, axis=1)
    m = lax.dot_general(z, w, (((1,), (1,)), ((), ())))
    d2 = (zz - 2.0 * m) + ww[None, :]
    dist = jnp.sqrt(jnp.maximum(d2, 0.0))
    vmin = jnp.min(dist, axis=1, keepdims=True)
    cols = lax.broadcasted_iota(jnp.int32, dist.shape, 1)
    idx = jnp.min(jnp.where(dist == vmin, cols, _NUM_CODES), axis=1)
    onehot = (idx[:, None] == cols).astype(jnp.float32)
    qz_ref[...] = lax.dot_general(onehot, w, (((1,), (0,)), ((), ())))
    part = jnp.sum(vmin * vmin)
    @pl.when(i == 0)
    def _():
        loss_ref[0, 0] = 0.0
    loss_ref[0, 0] += part


@functools.partial(jax.jit)
def kernel(z_e, W):
    z_tokens = jnp.transpose(z_e, (0, 2, 3, 1)).reshape(_TOKENS, _DIM)
    grid = (_TOKENS // _BLK,)
    qz, loss_sum = pl.pallas_call(
        _vq_body,
        grid=grid,
        in_specs=[
            pl.BlockSpec((_BLK, _DIM), lambda i: (i, 0)),
            pl.BlockSpec((_NUM_CODES, _DIM), lambda i: (0, 0)),
        ],
        out_specs=[
            pl.BlockSpec((_BLK, _DIM), lambda i: (i, 0)),
            pl.BlockSpec(memory_space=pltpu.SMEM, block_shape=(1, 1),
                         index_map=lambda i: (0, 0)),
        ],
        out_shape=[
            jax.ShapeDtypeStruct((_TOKENS, _DIM), jnp.float32),
            jax.ShapeDtypeStruct((1, 1), jnp.float32),
        ],
    )(z_tokens, W)
    z_q = qz.reshape(z_e.shape[0], z_e.shape[2], z_e.shape[3], _DIM)
    z_q = jnp.transpose(z_q, (0, 3, 1, 2))
    n = z_e.size
    loss = (loss_sum[0, 0] / n).astype(jnp.float32)
    return (z_q, loss, loss)
